# Initial kernel scaffold; baseline (speedup 1.0000x reference)
import jax
import jax.numpy as jnp
from jax.experimental import pallas as pl


def kernel(x, c1_w, c1_b, c2_w, c2_b, c3_w, c3_b, sc_w, sc_b):
    raise NotImplementedError("write your pallas kernel here")


# speedup vs baseline: 30.9854x; 30.9854x over previous
from functools import partial

import jax
import jax.numpy as jnp
from jax.experimental import pallas as pl
from jax.experimental.pallas import tpu as pltpu

VMEM_LIMIT = 32 * 1024 * 1024
MARGIN = 64
ROW_BLOCK = 8


def _fused_block_kernel(x_ref, c1w_ref, c1b_ref, c2p_ref, c2b_ref,
                        c3w_ref, c3b_ref, scw_ref, scb_ref,
                        o_ref, h1e_ref, *, H, W, pairs):
    Wp = W + 2
    RB = ROW_BLOCK
    nchunks = H // RB
    CH = RB * W

    h1e_ref[...] = jnp.zeros_like(h1e_ref)

    for c in range(nchunks):
        xc = x_ref[0, pl.ds(c * CH, CH), :].astype(jnp.bfloat16)
        h1 = jnp.dot(xc, c1w_ref[...], preferred_element_type=jnp.float32)
        h1 = jnp.maximum(h1 + c1b_ref[...], 0.0).astype(jnp.bfloat16)
        for r in range(RB):
            h = c * RB + r
            h1e_ref[pl.ds(MARGIN + (h + 1) * Wp + 1, W), :] = \
                h1[r * W:(r + 1) * W, :]

    for c in range(nchunks):
        p0 = MARGIN + (c * RB + 1) * Wp
        M = RB * Wp
        acc = None
        for i, (dta, dtb) in enumerate(pairs):
            lhs = jnp.concatenate(
                [h1e_ref[pl.ds(p0 + dta, M), :],
                 h1e_ref[pl.ds(p0 + dtb, M), :]], axis=1)
            d = jnp.dot(lhs, c2p_ref[i], preferred_element_type=jnp.float32)
            acc = d if acc is None else acc + d
        h2 = jnp.maximum(acc + c2b_ref[...], 0.0).astype(jnp.bfloat16)

        y = jnp.dot(h2, c3w_ref[...], preferred_element_type=jnp.float32)
        y = jnp.maximum(y + c3b_ref[...], 0.0)

        xc = x_ref[0, pl.ds(c * CH, CH), :].astype(jnp.bfloat16)
        s = jnp.dot(xc, scw_ref[...], preferred_element_type=jnp.float32)
        s = jnp.maximum(s + scb_ref[...], 0.0)

        for r in range(RB):
            h = c * RB + r
            o_ref[0, pl.ds(h * W, W), :] = (
                y[r * Wp + 1:r * Wp + 1 + W, :] + s[r * W:(r + 1) * W, :])


def kernel(x, c1_w, c1_b, c2_w, c2_b, c3_w, c3_b, sc_w, sc_b):
    N, H, W, Cin = x.shape
    Cmid = c1_w.shape[1]
    Cout = c3_w.shape[1]
    Wp = W + 2

    dts = [(di - 1) * Wp + (dj - 1) for di in range(3) for dj in range(3)]
    taps = [c2_w[di, dj * Cmid:(dj + 1) * Cmid, :]
            for di in range(3) for dj in range(3)]
    packed = jnp.stack([
        jnp.concatenate([taps[0], taps[1]], axis=0),
        jnp.concatenate([taps[2], taps[3]], axis=0),
        jnp.concatenate([taps[4], taps[5]], axis=0),
        jnp.concatenate([taps[6], taps[7]], axis=0),
        jnp.concatenate([taps[8], jnp.zeros_like(taps[8])], axis=0),
    ])
    pairs = [(dts[0], dts[1]), (dts[2], dts[3]), (dts[4], dts[5]),
             (dts[6], dts[7]), (dts[8], dts[8])]

    scratch_rows = (2 * MARGIN + (H + 2) * Wp + 15) // 16 * 16

    xf = x.reshape(N, H * W, Cin)
    out = pl.pallas_call(
        partial(_fused_block_kernel, H=H, W=W, pairs=pairs),
        out_shape=jax.ShapeDtypeStruct((N, H * W, Cout), jnp.float32),
        grid=(N,),
        in_specs=[
            pl.BlockSpec((1, H * W, Cin), lambda n: (n, 0, 0)),
            pl.BlockSpec((Cin, Cmid), lambda n: (0, 0)),
            pl.BlockSpec((1, Cmid), lambda n: (0, 0)),
            pl.BlockSpec((5, 2 * Cmid, Cmid), lambda n: (0, 0, 0)),
            pl.BlockSpec((1, Cmid), lambda n: (0, 0)),
            pl.BlockSpec((Cmid, Cout), lambda n: (0, 0)),
            pl.BlockSpec((1, Cout), lambda n: (0, 0)),
            pl.BlockSpec((Cin, Cout), lambda n: (0, 0)),
            pl.BlockSpec((1, Cout), lambda n: (0, 0)),
        ],
        out_specs=pl.BlockSpec((1, H * W, Cout), lambda n: (n, 0, 0)),
        scratch_shapes=[pltpu.VMEM((scratch_rows, Cmid), jnp.bfloat16)],
        compiler_params=pltpu.CompilerParams(
            dimension_semantics=("parallel",),
            vmem_limit_bytes=VMEM_LIMIT),
    )(xf, c1_w, c1_b.reshape(1, Cmid), packed, c2_b.reshape(1, Cmid),
      c3_w, c3_b.reshape(1, Cout), sc_w, sc_b.reshape(1, Cout))
    return out.reshape(N, H, W, Cout)
